# Initial kernel scaffold; baseline (speedup 1.0000x reference)
import jax
import jax.numpy as jnp
from jax.experimental import pallas as pl


def kernel(api_seq, perm_seq, api_table, perm_table):
    raise NotImplementedError("write your pallas kernel here")


# speedup vs baseline: 8.7845x; 8.7845x over previous
import functools

import jax
import jax.numpy as jnp
from jax import lax
from jax.experimental import pallas as pl
from jax.experimental.pallas import tpu as pltpu
from jax.experimental.pallas import tpu_sc as plsc

B = 4096
AL = 200
PLEN = 50
PPAD = 56
D = 128
NC = 2
NS = 16
W = NC * NS
R = B // W
NCH = D // 16


@functools.partial(
    pl.kernel,
    out_type=jax.ShapeDtypeStruct((B, 2 * D), jnp.float32),
    mesh=plsc.VectorSubcoreMesh(core_axis_name="c", subcore_axis_name="s"),
    scratch_types=[
        pltpu.VMEM((R * AL + 16,), jnp.int32),
        pltpu.VMEM((R * PPAD + 16,), jnp.int32),
        pltpu.VMEM((AL, D), jnp.float32),
        pltpu.VMEM((PLEN, D), jnp.float32),
        pltpu.VMEM((R, 2 * D), jnp.float32),
        pltpu.SemaphoreType.DMA,
        pltpu.SemaphoreType.DMA,
    ],
)
def _sc_embed(api_idx, perm_idx, api_table, perm_table, out,
              idx_a, idx_p, buf_a, buf_p, outb, sem_a, sem_p):
    wid = lax.axis_index("s") * NC + lax.axis_index("c")
    base = wid * R
    pltpu.sync_copy(api_idx.at[pl.ds(base * AL, R * AL)],
                    idx_a.at[pl.ds(0, R * AL)])
    pltpu.sync_copy(perm_idx.at[pl.ds(base * PPAD, R * PPAD)],
                    idx_p.at[pl.ds(0, R * PPAD)])
    lanes = lax.iota(jnp.int32, 16)

    def process_row(r, carry):
        off_a = r * AL
        off_p = r * PPAD
        ca1 = pltpu.async_copy(api_table.at[idx_a.at[pl.ds(off_a, 128)]],
                               buf_a.at[pl.ds(0, 128)], sem_a)
        ca2 = pltpu.async_copy(api_table.at[idx_a.at[pl.ds(off_a + 128, AL - 128)]],
                               buf_a.at[pl.ds(128, AL - 128)], sem_a)
        cp = pltpu.async_copy(perm_table.at[idx_p.at[pl.ds(off_p, PLEN)]],
                              buf_p, sem_p)

        one = jnp.ones(16, jnp.int32)
        zero = jnp.zeros(16, jnp.int32)

        def cnt_a(k, c):
            v = idx_a[pl.ds(off_a + k * 16, 16)]
            return c + jnp.where(v != 0, one, zero)
        c_a = lax.fori_loop(0, AL // 16, cnt_a, jnp.zeros(16, jnp.int32))
        v_tail = idx_a[pl.ds(off_a + (AL // 16) * 16, 16)]
        c_a = c_a + jnp.where((v_tail != 0) & (lanes < AL % 16), one, zero)

        def cnt_p(k, c):
            v = idx_p[pl.ds(off_p + k * 16, 16)]
            return c + jnp.where(v != 0, one, zero)
        c_p = lax.fori_loop(0, PLEN // 16, cnt_p, jnp.zeros(16, jnp.int32))
        v_tail_p = idx_p[pl.ds(off_p + (PLEN // 16) * 16, 16)]
        c_p = c_p + jnp.where((v_tail_p != 0) & (lanes < PLEN % 16), one, zero)

        n_a_s = c_a[0]
        n_p_s = c_p[0]
        for l in range(1, 16):
            n_a_s = n_a_s + c_a[l]
            n_p_s = n_p_s + c_p[l]
        n_a = jnp.maximum(jnp.full((16,), n_a_s, jnp.int32)
                          .astype(jnp.float32), 1.0)
        n_p = jnp.maximum(jnp.full((16,), n_p_s, jnp.int32)
                          .astype(jnp.float32), 1.0)

        ca1.wait()
        ca2.wait()
        cp.wait()

        def sum_a(i, accs):
            return tuple(a + buf_a[i, pl.ds(c * 16, 16)]
                         for c, a in enumerate(accs))
        acc_a = lax.fori_loop(0, AL, sum_a,
                              tuple(jnp.zeros(16, jnp.float32)
                                    for _ in range(NCH)))

        def sum_p(i, accs):
            return tuple(a + buf_p[i, pl.ds(c * 16, 16)]
                         for c, a in enumerate(accs))
        acc_p = lax.fori_loop(0, PLEN, sum_p,
                              tuple(jnp.zeros(16, jnp.float32)
                                    for _ in range(NCH)))

        inv_a = 1.0 / n_a
        inv_p = 1.0 / n_p
        for c in range(NCH):
            outb[r, pl.ds(c * 16, 16)] = acc_a[c] * inv_a
            outb[r, pl.ds(D + c * 16, 16)] = acc_p[c] * inv_p
        return carry

    lax.fori_loop(0, R, process_row, 0)
    pltpu.sync_copy(outb, out.at[pl.ds(base, R)])


def kernel(api_seq, perm_seq, api_table, perm_table):
    api_flat = api_seq.reshape(-1)
    perm_flat = jnp.pad(perm_seq, ((0, 0), (0, PPAD - PLEN))).reshape(-1)
    return _sc_embed(api_flat, perm_flat, api_table, perm_table)


# speedup vs baseline: 15.6067x; 1.7766x over previous
import functools

import jax
import jax.numpy as jnp
from jax import lax
from jax.experimental import pallas as pl
from jax.experimental.pallas import tpu as pltpu
from jax.experimental.pallas import tpu_sc as plsc

B = 4096
AL = 200
PLEN = 50
PPAD = 56
D = 128
NC = 2
NS = 16
W = NC * NS
R = B // W
NCH = D // 16


@functools.partial(
    pl.kernel,
    out_type=jax.ShapeDtypeStruct((B, 2 * D), jnp.float32),
    mesh=plsc.VectorSubcoreMesh(core_axis_name="c", subcore_axis_name="s"),
    scratch_types=[
        pltpu.VMEM((R * AL + 16,), jnp.int32),
        pltpu.VMEM((R * PPAD + 16,), jnp.int32),
        pltpu.VMEM((2 * AL, D), jnp.float32),
        pltpu.VMEM((2 * PLEN, D), jnp.float32),
        pltpu.VMEM((R, 2 * D), jnp.float32),
        pltpu.SemaphoreType.DMA,
        pltpu.SemaphoreType.DMA,
        pltpu.SemaphoreType.DMA,
        pltpu.SemaphoreType.DMA,
    ],
)
def _sc_embed(api_idx, perm_idx, api_table, perm_table, out,
              idx_a, idx_p, buf_a, buf_p, outb, sem_a0, sem_a1,
              sem_p0, sem_p1):
    wid = lax.axis_index("s") * NC + lax.axis_index("c")
    base = wid * R
    pltpu.sync_copy(api_idx.at[pl.ds(base * AL, R * AL)],
                    idx_a.at[pl.ds(0, R * AL)])
    pltpu.sync_copy(perm_idx.at[pl.ds(base * PPAD, R * PPAD)],
                    idx_p.at[pl.ds(0, R * PPAD)])
    lanes = lax.iota(jnp.int32, 16)
    sems_a = (sem_a0, sem_a1)
    sems_p = (sem_p0, sem_p1)

    def mk_copies(r, slot):
        off_a = r * AL
        off_p = r * PPAD
        sa = slot * AL
        sp = slot * PLEN
        return (
            (api_table.at[idx_a.at[pl.ds(off_a, 128)]],
             buf_a.at[pl.ds(sa, 128)], sems_a[slot]),
            (api_table.at[idx_a.at[pl.ds(off_a + 128, AL - 128)]],
             buf_a.at[pl.ds(sa + 128, AL - 128)], sems_a[slot]),
            (perm_table.at[idx_p.at[pl.ds(off_p, PLEN)]],
             buf_p.at[pl.ds(sp, PLEN)], sems_p[slot]),
        )

    def issue(r, slot):
        for src, dst, sem in mk_copies(r, slot):
            pltpu.async_copy(src, dst, sem)

    def drain(r, slot):
        for src, dst, sem in mk_copies(r, slot):
            pltpu.make_async_copy(src, dst, sem).wait()

    def process_row(r, slot):
        off_a = r * AL
        off_p = r * PPAD
        sa = slot * AL
        sp = slot * PLEN

        one = jnp.ones(16, jnp.int32)
        zero = jnp.zeros(16, jnp.int32)

        def cnt_a(k, c):
            v = idx_a[pl.ds(off_a + k * 16, 16)]
            return c + jnp.where(v != 0, one, zero)
        c_a = lax.fori_loop(0, AL // 16, cnt_a, jnp.zeros(16, jnp.int32),
                            unroll=4)
        v_tail = idx_a[pl.ds(off_a + (AL // 16) * 16, 16)]
        c_a = c_a + jnp.where((v_tail != 0) & (lanes < AL % 16), one, zero)

        def cnt_p(k, c):
            v = idx_p[pl.ds(off_p + k * 16, 16)]
            return c + jnp.where(v != 0, one, zero)
        c_p = lax.fori_loop(0, PLEN // 16, cnt_p, jnp.zeros(16, jnp.int32),
                            unroll=3)
        v_tail_p = idx_p[pl.ds(off_p + (PLEN // 16) * 16, 16)]
        c_p = c_p + jnp.where((v_tail_p != 0) & (lanes < PLEN % 16), one, zero)

        va = [c_a[l] for l in range(16)]
        vp = [c_p[l] for l in range(16)]
        while len(va) > 1:
            va = [va[i] + va[i + 1] for i in range(0, len(va), 2)]
            vp = [vp[i] + vp[i + 1] for i in range(0, len(vp), 2)]
        n_a = jnp.maximum(jnp.full((16,), va[0], jnp.int32)
                          .astype(jnp.float32), 1.0)
        n_p = jnp.maximum(jnp.full((16,), vp[0], jnp.int32)
                          .astype(jnp.float32), 1.0)

        drain(r, slot)

        def sum_a(i, accs):
            return tuple(a + buf_a[sa + i, pl.ds(c * 16, 16)]
                         for c, a in enumerate(accs))
        acc_a = lax.fori_loop(0, AL, sum_a,
                              tuple(jnp.zeros(16, jnp.float32)
                                    for _ in range(NCH)), unroll=4)

        def sum_p(i, accs):
            return tuple(a + buf_p[sp + i, pl.ds(c * 16, 16)]
                         for c, a in enumerate(accs))
        acc_p = lax.fori_loop(0, PLEN, sum_p,
                              tuple(jnp.zeros(16, jnp.float32)
                                    for _ in range(NCH)), unroll=4)

        inv_a = 1.0 / n_a
        inv_p = 1.0 / n_p
        for c in range(NCH):
            outb[r, pl.ds(c * 16, 16)] = acc_a[c] * inv_a
            outb[r, pl.ds(D + c * 16, 16)] = acc_p[c] * inv_p

    issue(0, 0)

    def body(g, carry):
        r0 = 2 * g
        issue(r0 + 1, 1)
        process_row(r0, 0)

        @pl.when(r0 + 2 < R)
        def _():
            issue(r0 + 2, 0)
        process_row(r0 + 1, 1)
        return carry

    lax.fori_loop(0, R // 2, body, 0)
    pltpu.sync_copy(outb, out.at[pl.ds(base, R)])


def kernel(api_seq, perm_seq, api_table, perm_table):
    api_flat = api_seq.reshape(-1)
    perm_flat = jnp.pad(perm_seq, ((0, 0), (0, PPAD - PLEN))).reshape(-1)
    return _sc_embed(api_flat, perm_flat, api_table, perm_table)
